# Initial kernel scaffold; baseline (speedup 1.0000x reference)
import functools
import math

import jax
import jax.numpy as jnp
from jax import lax
from jax.experimental import pallas as pl
from jax.experimental.pallas import tpu as pltpu
from jax.experimental.pallas import tpu_sc as plsc

_NUM_CODES = 1000000
_CODE_DIM = 16
_EMBED_DIM = 32
_HIDDEN = 128
_OUT = 64
_B = 4096
_T = 20

_N_IDX = _B * _T * _CODE_DIM
_ROWS = _B * _T
_FEAT = _CODE_DIM * _EMBED_DIM

_INFO = plsc.get_sparse_core_info()
_NC = _INFO.num_cores
_NS = _INFO.num_subcores
_NW = _NC * _NS
_PER_W = _N_IDX // _NW
_CHUNK = 2048
_STREAMS = _CHUNK // 128
_OUTER = _PER_W // _CHUNK


def _sc_gather(codes2d, table):
    mesh = plsc.VectorSubcoreMesh(core_axis_name="c", subcore_axis_name="s")

    @functools.partial(
        pl.kernel,
        mesh=mesh,
        out_type=jax.ShapeDtypeStruct((_N_IDX, _EMBED_DIM), jnp.float32),
        scratch_types=[
            pltpu.VMEM((_STREAMS, 128), jnp.int32),
            pltpu.VMEM((_CHUNK, _EMBED_DIM), jnp.float32),
            pltpu.SemaphoreType.DMA,
        ],
    )
    def k(codes_hbm, table_hbm, out_hbm, idx_v, rows_v, sem):
        wid = lax.axis_index("s") * _NC + lax.axis_index("c")

        def body(outer, carry):
            row0 = wid * (_OUTER * _STREAMS) + outer * _STREAMS
            el0 = wid * _PER_W + outer * _CHUNK
            pltpu.sync_copy(codes_hbm.at[pl.ds(row0, _STREAMS)], idx_v)
            cps = []
            for j in range(_STREAMS):
                cp = pltpu.async_copy(
                    table_hbm.at[idx_v.at[j]],
                    rows_v.at[pl.ds(j * 128, 128)],
                    sem,
                )
                cps.append(cp)
            for cp in cps:
                cp.wait()
            pltpu.sync_copy(rows_v, out_hbm.at[pl.ds(el0, _CHUNK)])
            return carry

        lax.fori_loop(0, _OUTER, body, 0)

    return k(codes2d, table)


_ROW_BLK = 1024


def _mlp_body(x_ref, w1_ref, b1_ref, gamma_ref, beta_ref, w2_ref, b2_ref, o_ref):
    x = x_ref[...]
    h = jnp.dot(x, w1_ref[...], preferred_element_type=jnp.float32) + b1_ref[...]
    h = 0.5 * h * (1.0 + lax.erf(h * (1.0 / math.sqrt(2.0))))
    mu = jnp.mean(h, axis=-1, keepdims=True)
    var = jnp.mean((h - mu) ** 2, axis=-1, keepdims=True)
    h = (h - mu) * lax.rsqrt(var + 1e-5)
    h = h * gamma_ref[...] + beta_ref[...]
    o_ref[...] = jnp.dot(h, w2_ref[...], preferred_element_type=jnp.float32) + b2_ref[...]


def _tc_mlp(x2d, W1, b1, gamma, beta, W2, b2):
    grid = (_ROWS // _ROW_BLK,)
    return pl.pallas_call(
        _mlp_body,
        grid=grid,
        in_specs=[
            pl.BlockSpec((_ROW_BLK, _FEAT), lambda i: (i, 0)),
            pl.BlockSpec((_FEAT, _HIDDEN), lambda i: (0, 0)),
            pl.BlockSpec((1, _HIDDEN), lambda i: (0, 0)),
            pl.BlockSpec((1, _HIDDEN), lambda i: (0, 0)),
            pl.BlockSpec((1, _HIDDEN), lambda i: (0, 0)),
            pl.BlockSpec((_HIDDEN, _OUT), lambda i: (0, 0)),
            pl.BlockSpec((1, _OUT), lambda i: (0, 0)),
        ],
        out_specs=pl.BlockSpec((_ROW_BLK, _OUT), lambda i: (i, 0)),
        out_shape=jax.ShapeDtypeStruct((_ROWS, _OUT), jnp.float32),
    )(x2d, W1, b1, gamma, beta, W2, b2)


def kernel(codes, table, W1, b1, gamma, beta, W2, b2):
    codes2d = codes.reshape(_N_IDX // 128, 128)
    rows = _sc_gather(codes2d, table)
    x2d = rows.reshape(_ROWS, _FEAT)
    out2d = _tc_mlp(
        x2d,
        W1,
        b1.reshape(1, _HIDDEN),
        gamma.reshape(1, _HIDDEN),
        beta.reshape(1, _HIDDEN),
        W2,
        b2.reshape(1, _OUT),
    )
    return out2d.reshape(_B, _T, _OUT)


# speedup vs baseline: 32.7896x; 32.7896x over previous
import functools
import math

import jax
import jax.numpy as jnp
from jax import lax
from jax.experimental import pallas as pl
from jax.experimental.pallas import tpu as pltpu
from jax.experimental.pallas import tpu_sc as plsc

_NUM_CODES = 1000000
_CODE_DIM = 16
_EMBED_DIM = 32
_HIDDEN = 128
_OUT = 64
_B = 4096
_T = 20

_N_IDX = _B * _T * _CODE_DIM
_ROWS = _B * _T
_FEAT = _CODE_DIM * _EMBED_DIM

_INFO = plsc.get_sparse_core_info()
_NC = _INFO.num_cores
_NS = _INFO.num_subcores
_NW = _NC * _NS
_PER_W = _N_IDX // _NW
_CHUNK = 2048
_STREAMS = _CHUNK // 128
_OUTER = _PER_W // _CHUNK


def _sc_gather(codes2d, table):
    mesh = plsc.VectorSubcoreMesh(core_axis_name="c", subcore_axis_name="s")

    @functools.partial(
        pl.kernel,
        mesh=mesh,
        out_type=jax.ShapeDtypeStruct((_N_IDX, _EMBED_DIM), jnp.float32),
        scratch_types=[
            pltpu.VMEM((_STREAMS, 128), jnp.int32),
            pltpu.VMEM((_CHUNK, _EMBED_DIM), jnp.float32),
            pltpu.SemaphoreType.DMA,
        ],
        compiler_params=pltpu.CompilerParams(use_tc_tiling_on_sc=False),
    )
    def k(codes_hbm, table_hbm, out_hbm, idx_v, rows_v, sem):
        wid = lax.axis_index("s") * _NC + lax.axis_index("c")

        def body(outer, carry):
            row0 = wid * (_OUTER * _STREAMS) + outer * _STREAMS
            el0 = wid * _PER_W + outer * _CHUNK
            pltpu.sync_copy(codes_hbm.at[pl.ds(row0, _STREAMS)], idx_v)
            cps = []
            for j in range(_STREAMS):
                cp = pltpu.async_copy(
                    table_hbm.at[idx_v.at[j]],
                    rows_v.at[pl.ds(j * 128, 128)],
                    sem,
                )
                cps.append(cp)
            for cp in cps:
                cp.wait()
            pltpu.sync_copy(rows_v, out_hbm.at[pl.ds(el0, _CHUNK)])
            return carry

        lax.fori_loop(0, _OUTER, body, 0)

    return k(codes2d, table)


_ROW_BLK = 1024


def _mlp_body(x_ref, w1_ref, b1_ref, gamma_ref, beta_ref, w2_ref, b2_ref, o_ref):
    x = x_ref[...]
    h = jnp.dot(x, w1_ref[...], preferred_element_type=jnp.float32) + b1_ref[...]
    h = 0.5 * h * (1.0 + lax.erf(h * (1.0 / math.sqrt(2.0))))
    mu = jnp.mean(h, axis=-1, keepdims=True)
    var = jnp.mean((h - mu) ** 2, axis=-1, keepdims=True)
    h = (h - mu) * lax.rsqrt(var + 1e-5)
    h = h * gamma_ref[...] + beta_ref[...]
    o_ref[...] = jnp.dot(h, w2_ref[...], preferred_element_type=jnp.float32) + b2_ref[...]


def _tc_mlp(x2d, W1, b1, gamma, beta, W2, b2):
    grid = (_ROWS // _ROW_BLK,)
    return pl.pallas_call(
        _mlp_body,
        grid=grid,
        in_specs=[
            pl.BlockSpec((_ROW_BLK, _FEAT), lambda i: (i, 0)),
            pl.BlockSpec((_FEAT, _HIDDEN), lambda i: (0, 0)),
            pl.BlockSpec((1, _HIDDEN), lambda i: (0, 0)),
            pl.BlockSpec((1, _HIDDEN), lambda i: (0, 0)),
            pl.BlockSpec((1, _HIDDEN), lambda i: (0, 0)),
            pl.BlockSpec((_HIDDEN, _OUT), lambda i: (0, 0)),
            pl.BlockSpec((1, _OUT), lambda i: (0, 0)),
        ],
        out_specs=pl.BlockSpec((_ROW_BLK, _OUT), lambda i: (i, 0)),
        out_shape=jax.ShapeDtypeStruct((_ROWS, _OUT), jnp.float32),
    )(x2d, W1, b1, gamma, beta, W2, b2)


def kernel(codes, table, W1, b1, gamma, beta, W2, b2):
    codes2d = codes.reshape(_N_IDX // 128, 128)
    rows = _sc_gather(codes2d, table)
    x2d = rows.reshape(_ROWS, _FEAT)
    out2d = _tc_mlp(
        x2d,
        W1,
        b1.reshape(1, _HIDDEN),
        gamma.reshape(1, _HIDDEN),
        beta.reshape(1, _HIDDEN),
        W2,
        b2.reshape(1, _OUT),
    )
    return out2d.reshape(_B, _T, _OUT)
